# Initial kernel scaffold; baseline (speedup 1.0000x reference)
import jax
import jax.numpy as jnp
from jax.experimental import pallas as pl


def kernel(x, cls_embedding, pos_embedding_global, pos_embedding_local):
    raise NotImplementedError("write your pallas kernel here")


# speedup vs baseline: 1.7679x; 1.7679x over previous
import jax
import jax.numpy as jnp
from jax.experimental import pallas as pl

_NUM_GLOBAL = 576
_NUM_LOCAL = 196


def _body(x_ref, cls_ref, pos_ref, out_ref):
    left = jnp.concatenate([cls_ref[...], x_ref[0]], axis=0)
    out_ref[0] = jnp.concatenate([left, pos_ref[...]], axis=1)


def kernel(x, cls_embedding, pos_embedding_global, pos_embedding_local):
    B, P, D = x.shape
    if P == _NUM_GLOBAL:
        pos = pos_embedding_global
    elif P == _NUM_LOCAL:
        pos = pos_embedding_local
    else:
        raise RuntimeError(f"Num patches {P} not matching")
    E = pos.shape[1]

    out = pl.pallas_call(
        _body,
        grid=(B,),
        in_specs=[
            pl.BlockSpec((1, P, D), lambda b: (b, 0, 0)),
            pl.BlockSpec((1, D), lambda b: (0, 0)),
            pl.BlockSpec((P + 1, E), lambda b: (0, 0)),
        ],
        out_specs=pl.BlockSpec((1, P + 1, D + E), lambda b: (b, 0, 0)),
        out_shape=jax.ShapeDtypeStruct((B, P + 1, D + E), x.dtype),
    )(x, cls_embedding, pos)
    return out
